# Initial kernel scaffold; baseline (speedup 1.0000x reference)
import functools

import jax
import jax.numpy as jnp
from jax.experimental import pallas as pl
from jax.experimental.pallas import tpu as pltpu
from jax.experimental.pallas import tpu_sc as plsc

EPS = 1e-5
LN_BLK = 4000
GATHER_WINDOW = 128


def _ln_body(table_ref, gamma_ref, beta_ref, out_ref):
    t = table_ref[...]
    mean = jnp.mean(t, axis=1, keepdims=True)
    c = t - mean
    var = jnp.mean(c * c, axis=1, keepdims=True)
    out_ref[...] = c * jax.lax.rsqrt(var + EPS) * gamma_ref[...] + beta_ref[...]


def _normalize_table(table, gamma, beta):
    v, d = table.shape
    blk = LN_BLK
    assert v % blk == 0
    return pl.pallas_call(
        _ln_body,
        grid=(v // blk,),
        in_specs=[
            pl.BlockSpec((blk, d), lambda i: (i, 0)),
            pl.BlockSpec((1, d), lambda i: (0, 0)),
            pl.BlockSpec((1, d), lambda i: (0, 0)),
        ],
        out_specs=pl.BlockSpec((blk, d), lambda i: (i, 0)),
        out_shape=jax.ShapeDtypeStruct((v, d), jnp.float32),
    )(table, gamma.reshape(1, d), beta.reshape(1, d))


def _sc_gather(table_n, idx_flat):
    b = idx_flat.shape[0]
    d = table_n.shape[1]
    w = GATHER_WINDOW
    assert b % w == 0
    idx2 = idx_flat.reshape(1, b)
    mesh = plsc.VectorSubcoreMesh(core_axis_name="core", subcore_axis_name="subcore")

    @functools.partial(
        pl.kernel,
        out_type=jax.ShapeDtypeStruct((b, d), jnp.float32),
        mesh=mesh,
    )
    def gather_kernel(table_hbm, i_hbm, o_hbm):
        def body(i_vmem, o_vmem):
            pltpu.sync_copy(table_hbm.at[i_vmem.at[0]], o_vmem)

        pltpu.emit_pipeline(
            body,
            grid=(b // w,),
            in_specs=[pl.BlockSpec((1, w), index_map=lambda i: (0, i))],
            out_specs=[pl.BlockSpec((w, d), index_map=lambda i: (i, 0))],
            core_axis_name=("core", "subcore"),
            dimension_semantics=(pltpu.PARALLEL,),
        )(i_hbm, o_hbm)

    return gather_kernel(table_n, idx2)


def kernel(x, table, gamma, beta):
    table_n = _normalize_table(table, gamma, beta)
    idx = x.reshape(-1).astype(jnp.int32)
    out = _sc_gather(table_n, idx)
    return out.reshape(x.shape + (table.shape[1],))


# speedup vs baseline: 3.8270x; 3.8270x over previous
import functools

import jax
import jax.numpy as jnp
from jax import lax
from jax.experimental import pallas as pl
from jax.experimental.pallas import tpu as pltpu
from jax.experimental.pallas import tpu_sc as plsc

EPS = 1e-5
LN_BLK = 4000
W = 128
NC, NS = 2, 16
NW = NC * NS


def _ln_body(table_ref, gamma_ref, beta_ref, out_ref):
    t = table_ref[...]
    mean = jnp.mean(t, axis=1, keepdims=True)
    c = t - mean
    var = jnp.mean(c * c, axis=1, keepdims=True)
    res = c * jax.lax.rsqrt(var + EPS) * gamma_ref[...] + beta_ref[...]
    out_ref[...] = jnp.concatenate([res, jnp.zeros_like(res)], axis=1)


def _normalize_table_padded(table, gamma, beta):
    v, d = table.shape
    blk = LN_BLK
    assert v % blk == 0
    return pl.pallas_call(
        _ln_body,
        grid=(v // blk,),
        in_specs=[
            pl.BlockSpec((blk, d), lambda i: (i, 0)),
            pl.BlockSpec((1, d), lambda i: (0, 0)),
            pl.BlockSpec((1, d), lambda i: (0, 0)),
        ],
        out_specs=pl.BlockSpec((blk, 2 * d), lambda i: (i, 0)),
        out_shape=jax.ShapeDtypeStruct((v, 2 * d), jnp.float32),
    )(table, gamma.reshape(1, d), beta.reshape(1, d))


def _sc_gather(table_p, idx_flat, d):
    b = idx_flat.shape[0]
    dp = table_p.shape[1]
    assert b % (W * NW) == 0
    per_w = b // NW
    steps = per_w // W
    mesh = plsc.VectorSubcoreMesh(core_axis_name="c", subcore_axis_name="s")

    @functools.partial(
        pl.kernel,
        out_type=jax.ShapeDtypeStruct((b, dp), jnp.float32),
        mesh=mesh,
        scratch_types=[
            pltpu.VMEM((W,), jnp.int32),
            pltpu.VMEM((W, dp), jnp.float32),
            pltpu.SemaphoreType.DMA,
        ],
    )
    def gather_kernel(table_hbm, i_hbm, o_hbm, idx_v, rows_v, sem):
        wid = lax.axis_index("s") * NC + lax.axis_index("c")
        w_base = wid * per_w

        @pl.loop(0, steps)
        def _(s):
            base = w_base + s * W
            pltpu.sync_copy(i_hbm.at[pl.ds(base, W)], idx_v)
            pltpu.async_copy(table_hbm.at[idx_v], rows_v, sem).wait()
            pltpu.sync_copy(rows_v, o_hbm.at[pl.ds(base, W)])

    return gather_kernel(table_p, idx_flat)


def kernel(x, table, gamma, beta):
    d = table.shape[1]
    table_p = _normalize_table_padded(table, gamma, beta)
    idx = x.reshape(-1).astype(jnp.int32)
    out_p = _sc_gather(table_p, idx, d)
    return out_p[:, :d].reshape(x.shape + (d,))
